# baseline (speedup 1.0000x reference)
import jax
import jax.numpy as jnp
from jax import lax
from jax.experimental import pallas as pl
from jax.experimental.pallas import tpu as pltpu
from jax.experimental.pallas import tpu_sc as plsc

BATCH = 16
MAX_LEN = 2048
D = 512
L = 16
NC, NS = 2, 16
NW = NC * NS
T_PER_W = MAX_LEN // NW
CHUNK_T = 4
CHUNK_ROWS = CHUNK_T * BATCH
NCHUNK = T_PER_W // CHUNK_T
PIECES = (16, 8, 4, 2, 1)


def _body(data_hbm, bs_hbm, zblk_hbm, out_hbm, bs_v, buf0, buf1,
          gsem0, gsem1, ssem0, ssem1):
    wid = lax.axis_index("s") * NC + lax.axis_index("c")
    t0 = wid * T_PER_W
    out0 = wid * (T_PER_W * BATCH)

    pltpu.sync_copy(bs_hbm, bs_v.at[pl.ds(0, MAX_LEN)])

    zf0 = pltpu.async_copy(zblk_hbm, buf0, ssem0)
    zf1 = pltpu.async_copy(zblk_hbm, buf1, ssem1)

    def acc_body(j, a):
        return a + bs_v[pl.ds(j * L, L)]

    acc = lax.fori_loop(0, t0 // L + T_PER_W // L, acc_body,
                        jnp.zeros((L,), jnp.int32))
    off_end = jnp.sum(acc)
    zf0.wait()
    zf1.wait()

    bufs = (buf0, buf1)
    gsems = (gsem0, gsem1)

    def gather_t(b, src0, buf, slot):
        for p in PIECES:
            pos = b & (~(2 * p - 1) & 31)

            @pl.when((b & p) != 0)
            def _(p=p, pos=pos):
                pltpu.async_copy(
                    data_hbm.at[pl.ds((src0 + pos) * D, p * D)],
                    buf.at[pl.ds((slot + pos) * D, p * D)],
                    gsems[_par[0]])

    def drain_t(b, buf, slot):
        for p in PIECES:
            pos = b & (~(2 * p - 1) & 31)

            @pl.when((b & p) != 0)
            def _(p=p, pos=pos):
                pltpu.make_async_copy(
                    data_hbm.at[pl.ds(0, p * D)],
                    buf.at[pl.ds((slot + pos) * D, p * D)],
                    gsems[_par[0]]).wait()

    ssems = (ssem0, ssem1)
    _par = [0]

    def drain_scatter(par):
        pltpu.make_async_copy(
            bufs[par], out_hbm.at[pl.ds(out0 * D, CHUNK_ROWS * D)],
            ssems[par]).wait()

    def chunk_offsets(c, off_after):
        bvec = bs_v[pl.ds(t0 + c * CHUNK_T, L)]
        bs_k = [bvec[k] for k in range(CHUNK_T)]
        offs = [None] * CHUNK_T
        off = off_after
        for k in range(CHUNK_T - 1, -1, -1):
            off = off - bs_k[k]
            offs[k] = off
        return bs_k, offs, off

    def issue_chunk(c, par, bs_k, offs, first):
        _par[0] = par

        @pl.when(jnp.logical_not(first))
        def _():
            drain_scatter(par)

        for k in range(CHUNK_T):
            gather_t(bs_k[k], offs[k], bufs[par], k * BATCH)

    def flush_chunk(c, par, bs_k):
        _par[0] = par
        for k in range(CHUNK_T):
            drain_t(bs_k[k], bufs[par], k * BATCH)
        pltpu.async_copy(
            bufs[par],
            out_hbm.at[pl.ds((out0 + c * CHUNK_ROWS) * D, CHUNK_ROWS * D)],
            ssems[par])

    def outer(i, off):
        c1 = (NCHUNK - 1) - 2 * i
        c0 = c1 - 1
        first = i == 0
        bs1, offs1, off = chunk_offsets(c1, off)
        issue_chunk(c1, 1, bs1, offs1, first)
        bs0, offs0, off = chunk_offsets(c0, off)
        issue_chunk(c0, 0, bs0, offs0, first)
        flush_chunk(c1, 1, bs1)
        flush_chunk(c0, 0, bs0)
        return off

    lax.fori_loop(0, NCHUNK // 2, outer, off_end)
    drain_scatter(0)
    drain_scatter(1)


def kernel(data, batch_sizes):
    bs32 = batch_sizes.astype(jnp.int32)
    zblk = jnp.zeros((CHUNK_ROWS * D,), jnp.float32)

    mesh = plsc.VectorSubcoreMesh(
        core_axis_name="c", subcore_axis_name="s", num_cores=NC,
        num_subcores=NS)
    out_flat = pl.kernel(
        _body,
        out_type=jax.ShapeDtypeStruct((MAX_LEN * BATCH * D,), jnp.float32),
        mesh=mesh,
        compiler_params=pltpu.CompilerParams(needs_layout_passes=False),
        scratch_types=[
            pltpu.VMEM((MAX_LEN + L,), jnp.int32),
            pltpu.VMEM((CHUNK_ROWS * D,), jnp.float32),
            pltpu.VMEM((CHUNK_ROWS * D,), jnp.float32),
            pltpu.SemaphoreType.DMA,
            pltpu.SemaphoreType.DMA,
            pltpu.SemaphoreType.DMA,
            pltpu.SemaphoreType.DMA,
        ],
    )(data.reshape(-1), bs32, zblk)
    return out_flat.reshape(MAX_LEN, BATCH, D)


# speedup vs baseline: 1.0385x; 1.0385x over previous
import jax
import jax.numpy as jnp
from jax import lax
from jax.experimental import pallas as pl
from jax.experimental.pallas import tpu as pltpu
from jax.experimental.pallas import tpu_sc as plsc

BATCH = 16
MAX_LEN = 2048
D = 512
L = 16
NC, NS = 2, 16
NW = NC * NS
T_PER_W = MAX_LEN // NW
CHUNK_T = 4
CHUNK_ROWS = CHUNK_T * BATCH
NCHUNK = T_PER_W // CHUNK_T
PIECES = (16, 8, 4, 2, 1)


def _body(data_hbm, bs_hbm, zblk_hbm, out_hbm, bs_v, buf0, buf1,
          gsem0, gsem1, ssem0, ssem1):
    wid = lax.axis_index("s") * NC + lax.axis_index("c")

    pltpu.sync_copy(bs_hbm, bs_v.at[pl.ds(0, MAX_LEN)])

    zf0 = pltpu.async_copy(zblk_hbm, buf0, ssem0)
    zf1 = pltpu.async_copy(zblk_hbm, buf1, ssem1)

    iota = lax.iota(jnp.int32, L)

    t_hi = 4 * (wid + NW * (NCHUNK - 1)) + CHUNK_T
    nfull = t_hi // L
    rem = t_hi % L

    def acc_body(j, a):
        return a + bs_v[pl.ds(j * L, L)]

    acc = lax.fori_loop(0, nfull, acc_body, jnp.zeros((L,), jnp.int32))
    tail = bs_v[pl.ds(nfull * L, L)]
    acc = acc + jnp.where(iota < rem, tail, 0)
    off_end = jnp.sum(acc)
    zf0.wait()
    zf1.wait()

    bufs = (buf0, buf1)
    gsems = (gsem0, gsem1)

    def gather_t(b, src0, buf, slot):
        for p in PIECES:
            pos = b & (~(2 * p - 1) & 31)

            @pl.when((b & p) != 0)
            def _(p=p, pos=pos):
                pltpu.async_copy(
                    data_hbm.at[pl.ds((src0 + pos) * D, p * D)],
                    buf.at[pl.ds((slot + pos) * D, p * D)],
                    gsems[_par[0]])

    def drain_t(b, buf, slot):
        for p in PIECES:
            pos = b & (~(2 * p - 1) & 31)

            @pl.when((b & p) != 0)
            def _(p=p, pos=pos):
                pltpu.make_async_copy(
                    data_hbm.at[pl.ds(0, p * D)],
                    buf.at[pl.ds((slot + pos) * D, p * D)],
                    gsems[_par[0]]).wait()

    ssems = (ssem0, ssem1)
    _par = [0]

    def drain_scatter(par):
        pltpu.make_async_copy(
            bufs[par], out_hbm.at[pl.ds(0, CHUNK_ROWS * D)],
            ssems[par]).wait()

    def chunk_offsets(g, off_after):
        bvec = bs_v[pl.ds(CHUNK_T * g, L)]
        bs_k = [bvec[k] for k in range(CHUNK_T)]
        offs = [None] * CHUNK_T
        off = off_after
        for k in range(CHUNK_T - 1, -1, -1):
            off = off - bs_k[k]
            offs[k] = off
        return bs_k, offs, off

    def gap_sum(g):
        lo = jnp.maximum(CHUNK_T * g - (CHUNK_T * NW - CHUNK_T), 0)

        def gap_body(j, a):
            return a + bs_v[pl.ds(lo + j * L, L)]

        nv = (CHUNK_T * NW) // L
        return jnp.sum(lax.fori_loop(0, nv, gap_body,
                                     jnp.zeros((L,), jnp.int32)))

    def issue_chunk(par, bs_k, offs, first):
        _par[0] = par

        @pl.when(jnp.logical_not(first))
        def _():
            drain_scatter(par)

        for k in range(CHUNK_T):
            gather_t(bs_k[k], offs[k], bufs[par], k * BATCH)

    def flush_chunk(g, par, bs_k):
        _par[0] = par
        for k in range(CHUNK_T):
            drain_t(bs_k[k], bufs[par], k * BATCH)
        pltpu.async_copy(
            bufs[par],
            out_hbm.at[pl.ds(g * (CHUNK_ROWS * D), CHUNK_ROWS * D)],
            ssems[par])

    def outer(i, e1):
        j1 = (NCHUNK - 1) - 2 * i
        g1 = wid + NW * j1
        g0 = g1 - NW
        first = i == 0
        bs1, offs1, _ = chunk_offsets(g1, e1)
        issue_chunk(1, bs1, offs1, first)
        e0 = e1 - gap_sum(g1)
        bs0, offs0, _ = chunk_offsets(g0, e0)
        issue_chunk(0, bs0, offs0, first)
        flush_chunk(g1, 1, bs1)
        flush_chunk(g0, 0, bs0)
        return e0 - gap_sum(g0)

    lax.fori_loop(0, NCHUNK // 2, outer, off_end)
    drain_scatter(0)
    drain_scatter(1)


def kernel(data, batch_sizes):
    bs32 = batch_sizes.astype(jnp.int32)
    zblk = jnp.zeros((CHUNK_ROWS * D,), jnp.float32)

    mesh = plsc.VectorSubcoreMesh(
        core_axis_name="c", subcore_axis_name="s", num_cores=NC,
        num_subcores=NS)
    out_flat = pl.kernel(
        _body,
        out_type=jax.ShapeDtypeStruct((MAX_LEN * BATCH * D,), jnp.float32),
        mesh=mesh,
        compiler_params=pltpu.CompilerParams(needs_layout_passes=False),
        scratch_types=[
            pltpu.VMEM((MAX_LEN + L,), jnp.int32),
            pltpu.VMEM((CHUNK_ROWS * D,), jnp.float32),
            pltpu.VMEM((CHUNK_ROWS * D,), jnp.float32),
            pltpu.SemaphoreType.DMA,
            pltpu.SemaphoreType.DMA,
            pltpu.SemaphoreType.DMA,
            pltpu.SemaphoreType.DMA,
        ],
    )(data.reshape(-1), bs32, zblk)
    return out_flat.reshape(MAX_LEN, BATCH, D)
